# Initial kernel scaffold; baseline (speedup 1.0000x reference)
import jax
import jax.numpy as jnp
from jax.experimental import pallas as pl


def kernel(x, edge_index, edge_type, weights):
    raise NotImplementedError("write your pallas kernel here")


# speedup vs baseline: 2.4195x; 2.4195x over previous
import functools

import jax
import jax.numpy as jnp
from jax import lax
from jax.experimental import pallas as pl
from jax.experimental.pallas import tpu as pltpu
from jax.experimental.pallas import tpu_sc as plsc

N_NODES_ = 10000
N_EDGES_ = 320000
N_CH_ = 128

NC = 2
NS = 16
NW = NC * NS
EPW = N_EDGES_ // NW
C = 80
NCHUNK = EPW // C


def _normalize_rows_tc(x):
    def body(x_ref, o_ref):
        v = x_ref[...]
        o_ref[...] = v * lax.rsqrt(jnp.sum(v * v, axis=1, keepdims=True))

    return pl.pallas_call(
        body,
        out_shape=jax.ShapeDtypeStruct(x.shape, x.dtype),
    )(x)


@functools.partial(
    pl.kernel,
    out_type=jax.ShapeDtypeStruct((N_EDGES_,), jnp.float32),
    mesh=plsc.VectorSubcoreMesh(core_axis_name="c", subcore_axis_name="s"),
    compiler_params=pltpu.CompilerParams(needs_layout_passes=False),
    scratch_types=dict(
        idx_s=pltpu.VMEM((EPW,), jnp.int32),
        idx_d=pltpu.VMEM((EPW,), jnp.int32),
        idx_r=pltpu.VMEM((EPW,), jnp.int32),
        s_rows=pltpu.VMEM((C, N_CH_), jnp.float32),
        o_rows=pltpu.VMEM((C, N_CH_), jnp.float32),
        r_rows=pltpu.VMEM((C, N_CH_), jnp.float32),
        out_v=pltpu.VMEM((EPW,), jnp.float32),
        sem_s=pltpu.SemaphoreType.DMA,
        sem_o=pltpu.SemaphoreType.DMA,
        sem_r=pltpu.SemaphoreType.DMA,
    ),
)
def _distmult_sc(xn_hbm, src_hbm, dst_hbm, rel_hbm, w_hbm, out_hbm,
                 idx_s, idx_d, idx_r, s_rows, o_rows, r_rows, out_v,
                 sem_s, sem_o, sem_r):
    wid = lax.axis_index("s") * NC + lax.axis_index("c")
    base = wid * EPW
    pltpu.sync_copy(src_hbm.at[pl.ds(base, EPW)], idx_s)
    pltpu.sync_copy(dst_hbm.at[pl.ds(base, EPW)], idx_d)
    pltpu.sync_copy(rel_hbm.at[pl.ds(base, EPW)], idx_r)

    def chunk_body(i, carry):
        off = i * C
        cp_s = pltpu.async_copy(xn_hbm.at[idx_s.at[pl.ds(off, C)]], s_rows, sem_s)
        cp_o = pltpu.async_copy(xn_hbm.at[idx_d.at[pl.ds(off, C)]], o_rows, sem_o)
        cp_r = pltpu.async_copy(w_hbm.at[idx_r.at[pl.ds(off, C)]], r_rows, sem_r)
        cp_s.wait()
        cp_o.wait()
        cp_r.wait()

        lane = lax.iota(jnp.int32, 16)

        def group_body(g, c2):
            vec = jnp.zeros((16,), jnp.float32)
            for m in range(16):
                e = g * 16 + m
                acc = (s_rows[e, pl.ds(0, 16)] * r_rows[e, pl.ds(0, 16)]
                       * o_rows[e, pl.ds(0, 16)])
                for k in range(1, N_CH_ // 16):
                    acc = acc + (s_rows[e, pl.ds(k * 16, 16)]
                                 * r_rows[e, pl.ds(k * 16, 16)]
                                 * o_rows[e, pl.ds(k * 16, 16)])
                vec = jnp.where(lane == m, jnp.sum(acc), vec)
            out_v[pl.ds(off + g * 16, 16)] = vec
            return c2

        lax.fori_loop(0, C // 16, group_body, 0)
        return carry

    lax.fori_loop(0, NCHUNK, chunk_body, 0)
    pltpu.sync_copy(out_v, out_hbm.at[pl.ds(base, EPW)])


def kernel(x, edge_index, edge_type, weights):
    xn = _normalize_rows_tc(x)
    src = edge_index[0, :].astype(jnp.int32)
    dst = edge_index[1, :].astype(jnp.int32)
    rel = edge_type.astype(jnp.int32)
    return _distmult_sc(xn, src, dst, rel, weights)


# speedup vs baseline: 3.1239x; 1.2912x over previous
import functools

import jax
import jax.numpy as jnp
from jax import lax
from jax.experimental import pallas as pl
from jax.experimental.pallas import tpu as pltpu
from jax.experimental.pallas import tpu_sc as plsc

N_NODES_ = 10000
N_EDGES_ = 320000
N_CH_ = 128

NC = 2
NS = 16
NW = NC * NS
EPW = N_EDGES_ // NW
C = 80
NCHUNK = EPW // C


def _normalize_rows_tc(x):
    def body(x_ref, o_ref):
        v = x_ref[...]
        o_ref[...] = v * lax.rsqrt(jnp.sum(v * v, axis=1, keepdims=True))

    return pl.pallas_call(
        body,
        out_shape=jax.ShapeDtypeStruct(x.shape, x.dtype),
    )(x)


@functools.partial(
    pl.kernel,
    out_type=jax.ShapeDtypeStruct((N_EDGES_,), jnp.float32),
    mesh=plsc.VectorSubcoreMesh(core_axis_name="c", subcore_axis_name="s"),
    compiler_params=pltpu.CompilerParams(needs_layout_passes=False),
    scratch_types=dict(
        idx_s=pltpu.VMEM((EPW,), jnp.int32),
        idx_d=pltpu.VMEM((EPW,), jnp.int32),
        idx_r=pltpu.VMEM((EPW,), jnp.int32),
        s_rows=[pltpu.VMEM((C, N_CH_), jnp.float32) for _ in range(2)],
        o_rows=[pltpu.VMEM((C, N_CH_), jnp.float32) for _ in range(2)],
        r_rows=[pltpu.VMEM((C, N_CH_), jnp.float32) for _ in range(2)],
        out_v=pltpu.VMEM((EPW,), jnp.float32),
        sem_s=[pltpu.SemaphoreType.DMA for _ in range(2)],
        sem_o=[pltpu.SemaphoreType.DMA for _ in range(2)],
        sem_r=[pltpu.SemaphoreType.DMA for _ in range(2)],
    ),
)
def _distmult_sc(xn_hbm, src_hbm, dst_hbm, rel_hbm, w_hbm, out_hbm,
                 idx_s, idx_d, idx_r, s_rows, o_rows, r_rows, out_v,
                 sem_s, sem_o, sem_r):
    wid = lax.axis_index("s") * NC + lax.axis_index("c")
    base = wid * EPW
    pltpu.sync_copy(src_hbm.at[pl.ds(base, EPW)], idx_s)
    pltpu.sync_copy(dst_hbm.at[pl.ds(base, EPW)], idx_d)
    pltpu.sync_copy(rel_hbm.at[pl.ds(base, EPW)], idx_r)

    lane = lax.iota(jnp.int32, 16)

    def fire(ci, b):
        off = ci * C
        pltpu.async_copy(xn_hbm.at[idx_s.at[pl.ds(off, C)]], s_rows[b], sem_s[b])
        pltpu.async_copy(xn_hbm.at[idx_d.at[pl.ds(off, C)]], o_rows[b], sem_o[b])
        pltpu.async_copy(w_hbm.at[idx_r.at[pl.ds(off, C)]], r_rows[b], sem_r[b])

    def drain(ci, b):
        off = ci * C
        pltpu.make_async_copy(
            xn_hbm.at[idx_s.at[pl.ds(off, C)]], s_rows[b], sem_s[b]).wait()
        pltpu.make_async_copy(
            xn_hbm.at[idx_d.at[pl.ds(off, C)]], o_rows[b], sem_o[b]).wait()
        pltpu.make_async_copy(
            w_hbm.at[idx_r.at[pl.ds(off, C)]], r_rows[b], sem_r[b]).wait()

    def compute(ci, b):
        off = ci * C
        sb, rb, ob = s_rows[b], r_rows[b], o_rows[b]

        def group_body(g, c2):
            vec = jnp.zeros((16,), jnp.float32)
            for m in range(16):
                e = g * 16 + m
                acc = sb[e, pl.ds(0, 16)] * rb[e, pl.ds(0, 16)] * ob[e, pl.ds(0, 16)]
                for k in range(1, N_CH_ // 16):
                    acc = acc + (sb[e, pl.ds(k * 16, 16)]
                                 * rb[e, pl.ds(k * 16, 16)]
                                 * ob[e, pl.ds(k * 16, 16)])
                vec = jnp.where(lane == m, jnp.sum(acc), vec)
            out_v[pl.ds(off + g * 16, 16)] = vec
            return c2

        lax.fori_loop(0, C // 16, group_body, 0)

    fire(0, 0)

    def pair_body(i, carry):
        c0 = 2 * i
        fire(c0 + 1, 1)
        drain(c0, 0)
        compute(c0, 0)
        fire(c0 + 2, 0)
        drain(c0 + 1, 1)
        compute(c0 + 1, 1)
        return carry

    lax.fori_loop(0, (NCHUNK - 1) // 2, pair_body, 0)
    drain(NCHUNK - 1, 0)
    compute(NCHUNK - 1, 0)

    pltpu.sync_copy(out_v, out_hbm.at[pl.ds(base, EPW)])


def kernel(x, edge_index, edge_type, weights):
    xn = _normalize_rows_tc(x)
    src = edge_index[0, :].astype(jnp.int32)
    dst = edge_index[1, :].astype(jnp.int32)
    rel = edge_type.astype(jnp.int32)
    return _distmult_sc(xn, src, dst, rel, weights)
